# baseline (speedup 1.0000x reference)
import functools

import jax
import jax.numpy as jnp
from jax import lax
from jax.experimental import pallas as pl
from jax.experimental.pallas import tpu as pltpu
from jax.experimental.pallas import tpu_sc as plsc

NUM_UNITS = 64
TABLE_ROWS = 257
SEQ = 2048
B = SEQ * SEQ
NC, NS, L = 2, 16, 16
NW = NC * NS
BLOCK = 512
BLK_PER_ROW = SEQ // BLOCK
NCHUNK = 4
CHUNK_I = SEQ // NCHUNK
PER_W = CHUNK_I * SEQ // NW
ROWS_PER_W = CHUNK_I // NW
N_BLOCKS = PER_W // BLOCK
GROUPS = BLOCK // L


def _make_sc_gather():
    mesh = plsc.VectorSubcoreMesh(core_axis_name="c", subcore_axis_name="s")

    @functools.partial(
        pl.kernel,
        mesh=mesh,
        compiler_params=pltpu.CompilerParams(
            needs_layout_passes=False, use_tc_tiling_on_sc=False
        ),
        out_type=jax.ShapeDtypeStruct((CHUNK_I, SEQ, NUM_UNITS), jnp.float32),
        scratch_types=[
            pltpu.VMEM((TABLE_ROWS * NUM_UNITS,), jnp.float32),
            pltpu.VMEM((BLOCK,), jnp.int32),
            pltpu.VMEM((BLOCK,), jnp.int32),
            pltpu.VMEM((BLOCK, NUM_UNITS), jnp.float32),
            pltpu.VMEM((BLOCK, NUM_UNITS), jnp.float32),
            pltpu.SemaphoreType.DMA,
            pltpu.SemaphoreType.DMA,
        ],
    )
    def sc_gather(fm_hbm, table_hbm, out_hbm, table_v, idx0, idx1, rows0,
                  rows1, sem0, sem1):
        wid = lax.axis_index("s") * NC + lax.axis_index("c")
        base = wid * PER_W
        row_base = wid * ROWS_PER_W
        pltpu.sync_copy(table_hbm, table_v)

        def expand(idx_v, rows_v, blk):
            off = base + blk * BLOCK
            pltpu.sync_copy(fm_hbm.at[pl.ds(off, BLOCK)], idx_v)

            def group_body(g, c):
                iv = idx_v[pl.ds(g * L, L)] * NUM_UNITS
                for r in range(L):
                    src = iv[r]
                    b = g * L + r
                    vals = [
                        table_v[pl.ds(src + k, L)]
                        for k in range(0, NUM_UNITS, L)
                    ]
                    for k, v in zip(range(0, NUM_UNITS, L), vals):
                        rows_v[b, pl.ds(k, L)] = v
                return c

            lax.fori_loop(0, GROUPS, group_body, 0)

        def out_slice(blk):
            i = row_base + blk // BLK_PER_ROW
            j0 = (blk % BLK_PER_ROW) * BLOCK
            return out_hbm.at[i, pl.ds(j0, BLOCK)]

        def pair_body(i, carry):
            blk0 = 2 * i
            blk1 = blk0 + 1

            @pl.when(i > 0)
            def _():
                pltpu.make_async_copy(rows0, out_slice(blk0), sem0).wait()

            expand(idx0, rows0, blk0)
            pltpu.async_copy(rows0, out_slice(blk0), sem0)

            @pl.when(i > 0)
            def _():
                pltpu.make_async_copy(rows1, out_slice(blk1), sem1).wait()

            expand(idx1, rows1, blk1)
            pltpu.async_copy(rows1, out_slice(blk1), sem1)
            return carry

        lax.fori_loop(0, N_BLOCKS // 2, pair_body, 0)
        pltpu.make_async_copy(rows0, out_slice(0), sem0).wait()
        pltpu.make_async_copy(rows1, out_slice(1), sem1).wait()

    return sc_gather


_sc_gather = _make_sc_gather()


def kernel(final_mat, embeddings_table):
    fm = final_mat.reshape(B).astype(jnp.int32)
    table = embeddings_table.reshape(-1)
    per_chunk = CHUNK_I * SEQ
    out = jnp.zeros((SEQ, SEQ, NUM_UNITS), jnp.float32)
    for c in range(NCHUNK):
        chunk = _sc_gather(
            lax.dynamic_slice(fm, (c * per_chunk,), (per_chunk,)), table
        )
        out = lax.dynamic_update_slice(out, chunk, (c * CHUNK_I, 0, 0))
    return out


# speedup vs baseline: 1.5333x; 1.5333x over previous
import functools

import jax
import jax.numpy as jnp
from jax import lax
from jax.experimental import pallas as pl
from jax.experimental.pallas import tpu as pltpu
from jax.experimental.pallas import tpu_sc as plsc

NUM_UNITS = 64
TABLE_ROWS = 257
SEQ = 2048
B = SEQ * SEQ
NC, NS, L = 2, 16, 16
NW = NC * NS
BLOCK = 512
BLK_PER_ROW = SEQ // BLOCK
PER_W = B // NW
ROWS_PER_W = SEQ // NW
N_BLOCKS = PER_W // BLOCK
GROUPS = BLOCK // L
KS = tuple(range(0, NUM_UNITS, L))


def _make_sc_gather():
    mesh = plsc.VectorSubcoreMesh(core_axis_name="c", subcore_axis_name="s")

    @functools.partial(
        pl.kernel,
        mesh=mesh,
        compiler_params=pltpu.CompilerParams(
            needs_layout_passes=False, use_tc_tiling_on_sc=False
        ),
        out_type=jax.ShapeDtypeStruct((SEQ, SEQ, NUM_UNITS), jnp.float32),
        scratch_types=[
            pltpu.VMEM((TABLE_ROWS * NUM_UNITS,), jnp.float32),
            pltpu.VMEM((BLOCK,), jnp.int32),
            pltpu.VMEM((BLOCK,), jnp.int32),
            pltpu.VMEM((BLOCK, NUM_UNITS), jnp.float32),
            pltpu.VMEM((BLOCK, NUM_UNITS), jnp.float32),
            pltpu.SemaphoreType.DMA,
            pltpu.SemaphoreType.DMA,
            pltpu.SemaphoreType.DMA,
            pltpu.SemaphoreType.DMA,
        ],
    )
    def sc_gather(fm_hbm, table_hbm, out_hbm, table_v, idx0, idx1, rows0,
                  rows1, sem0, sem1, isem0, isem1):
        wid = lax.axis_index("s") * NC + lax.axis_index("c")
        base = wid * PER_W
        row_base = wid * ROWS_PER_W
        pltpu.sync_copy(table_hbm, table_v)

        def idx_src(blk):
            blk = jnp.minimum(blk, N_BLOCKS - 1)
            return fm_hbm.at[pl.ds(base + blk * BLOCK, BLOCK)]

        def expand(idx_v, rows_v):
            def group_body(g, c):
                iv = idx_v[pl.ds(g * L, L)] * NUM_UNITS
                for r in range(0, L, 2):
                    sa = iv[r]
                    sb = iv[r + 1]
                    ba = g * L + r
                    bb = ba + 1
                    va = [table_v[pl.ds(sa + k, L)] for k in KS]
                    vb = [table_v[pl.ds(sb + k, L)] for k in KS]
                    for k, v in zip(KS, va):
                        rows_v[ba, pl.ds(k, L)] = v
                    for k, v in zip(KS, vb):
                        rows_v[bb, pl.ds(k, L)] = v
                return c

            lax.fori_loop(0, GROUPS, group_body, 0)

        def out_slice(blk):
            i = row_base + blk // BLK_PER_ROW
            j0 = (blk % BLK_PER_ROW) * BLOCK
            return out_hbm.at[i, pl.ds(j0, BLOCK)]

        pltpu.async_copy(idx_src(0), idx0, isem0)

        def pair_body(i, carry):
            blk0 = 2 * i
            blk1 = blk0 + 1

            pltpu.async_copy(idx_src(blk1), idx1, isem1)
            pltpu.make_async_copy(idx_src(blk0), idx0, isem0).wait()

            @pl.when(i > 0)
            def _():
                pltpu.make_async_copy(rows0, out_slice(blk0), sem0).wait()

            expand(idx0, rows0)
            pltpu.async_copy(rows0, out_slice(blk0), sem0)
            pltpu.async_copy(idx_src(blk0 + 2), idx0, isem0)
            pltpu.make_async_copy(idx_src(blk1), idx1, isem1).wait()

            @pl.when(i > 0)
            def _():
                pltpu.make_async_copy(rows1, out_slice(blk1), sem1).wait()

            expand(idx1, rows1)
            pltpu.async_copy(rows1, out_slice(blk1), sem1)
            return carry

        lax.fori_loop(0, N_BLOCKS // 2, pair_body, 0)
        pltpu.make_async_copy(idx_src(0), idx0, isem0).wait()
        pltpu.make_async_copy(rows0, out_slice(0), sem0).wait()
        pltpu.make_async_copy(rows1, out_slice(1), sem1).wait()

    return sc_gather


_sc_gather = _make_sc_gather()


def kernel(final_mat, embeddings_table):
    fm = final_mat.reshape(B).astype(jnp.int32)
    return _sc_gather(fm, embeddings_table.reshape(-1))
